# baseline (speedup 1.0000x reference)
import functools

import jax
import jax.numpy as jnp
import numpy as np
from jax import lax
from jax.experimental import pallas as pl
from jax.experimental.pallas import tpu as pltpu
from jax.experimental.pallas import tpu_sc as plsc

_MASK_RATIO = 0.2
_MASK_KEY = 42

_LANES = 16
_NC, _NS = 2, 16


@functools.lru_cache(maxsize=None)
def _mask_indices(batch: int, time: int) -> np.ndarray:
    num_masks = int(_MASK_RATIO * time)

    def draw():
        keys = jax.random.split(jax.random.key(_MASK_KEY), batch)
        return jax.vmap(
            lambda k: jax.random.choice(k, time, shape=(num_masks,), replace=False)
        )(keys)

    with jax.ensure_compile_time_eval():
        try:
            cpu = jax.local_devices(backend="cpu")[0]
            with jax.default_device(cpu):
                idx = draw()
        except RuntimeError:
            idx = draw()
        idx = np.asarray(jax.device_get(idx), dtype=np.int32)
    pad = (-num_masks) % _LANES
    if pad:
        idx = np.concatenate([idx, np.repeat(idx[:, -1:], pad, axis=1)], axis=1)
    return idx


def _build_mask_sc(idx: jax.Array, batch: int, t_pad: int) -> jax.Array:
    n_pad = idx.shape[1]
    ones_row = jnp.ones((t_pad,), jnp.float32)
    mesh = plsc.VectorSubcoreMesh(core_axis_name="c", subcore_axis_name="s")

    @functools.partial(
        pl.kernel,
        out_type=jax.ShapeDtypeStruct((batch, t_pad), jnp.float32),
        mesh=mesh,
        compiler_params=pltpu.CompilerParams(needs_layout_passes=False),
        scratch_types=[
            pltpu.VMEM((n_pad,), jnp.int32),
            pltpu.VMEM((t_pad,), jnp.float32),
            pltpu.SemaphoreType.DMA,
            pltpu.SemaphoreType.DMA,
        ],
    )
    def sc_kernel(idx_hbm, ones_hbm, mask_hbm, idx_v, row_v, sem_i, sem_r):
        wid = lax.axis_index("s") * _NC + lax.axis_index("c")

        zeros16 = jnp.zeros((_LANES,), jnp.float32)

        cp_i = pltpu.async_copy(idx_hbm.at[wid], idx_v, sem_i)
        cp_r = pltpu.async_copy(ones_hbm, row_v, sem_r)
        cp_i.wait()
        cp_r.wait()

        for j in range(n_pad // _LANES):
            iv = idx_v[pl.ds(j * _LANES, _LANES)]
            plsc.store_scatter(row_v, [iv], zeros16)

        pltpu.sync_copy(row_v, mask_hbm.at[wid])

    return sc_kernel(idx, ones_row)


def _apply_mask_tc(x: jax.Array, mask: jax.Array) -> jax.Array:
    batch, freq, time = x.shape
    t_pad = mask.shape[-1]
    mask3 = mask.reshape(batch, 1, t_pad)

    bb = 16

    def body(x_ref, m_ref, o_ref):
        o_ref[...] = x_ref[...] * m_ref[:, :, :time]

    return pl.pallas_call(
        body,
        grid=(batch // bb,),
        in_specs=[
            pl.BlockSpec((bb, freq, time), lambda b: (b, 0, 0)),
            pl.BlockSpec((bb, 1, t_pad), lambda b: (b, 0, 0)),
        ],
        out_specs=pl.BlockSpec((bb, freq, time), lambda b: (b, 0, 0)),
        out_shape=jax.ShapeDtypeStruct((batch, freq, time), jnp.float32),
        compiler_params=pltpu.CompilerParams(
            vmem_limit_bytes=100 * 1024 * 1024,
        ),
    )(x, mask3)


def kernel(input_spec):
    batch, freq, time = input_spec.shape
    t_pad = time + ((-time) % _LANES)
    idx = jnp.asarray(_mask_indices(batch, time))
    mask = _build_mask_sc(idx, batch, t_pad)
    return _apply_mask_tc(input_spec, mask)
